# Initial kernel scaffold; baseline (speedup 1.0000x reference)
import jax
import jax.numpy as jnp
from jax.experimental import pallas as pl


def kernel(x, y):
    raise NotImplementedError("write your pallas kernel here")


# speedup vs baseline: 6.0847x; 6.0847x over previous
import jax
import jax.numpy as jnp
from jax.experimental import pallas as pl
from jax.experimental.pallas import tpu as pltpu

GAMMA_ = 1.0
LARGE_ = 1e30
B_BLK = 8
N_ = 512
M_ = 512
D_ = 64


def _sdtw_kernel(x_ref, y_ref, out_ref, dsk0_ref, dsk1_ref, dsk2_ref,
                 dsk3_ref, dt_ref):
    dsk_refs = (dsk0_ref, dsk1_ref, dsk2_ref, dsk3_ref)
    ones = jnp.ones((N_, 1), jnp.float32)
    il1 = jax.lax.broadcasted_iota(jnp.int32, (1, 128), 1)

    for b in range(B_BLK):
        xb = x_ref[b]
        yb = y_ref[b]
        x2 = jnp.sum(xb * xb, axis=1, keepdims=True)
        y2 = jnp.sum(yb * yb, axis=1, keepdims=True)
        lhs = jnp.concatenate([-2.0 * yb, y2, ones], axis=1)
        rhs = jnp.concatenate([xb, ones, x2], axis=1)
        dt_ref[...] = jax.lax.dot_general(
            lhs, rhs, (((1,), (1,)), ((), ())),
            preferred_element_type=jnp.float32)
        for c in range(4):
            a = dt_ref[:, 128 * c:128 * (c + 1)]
            for k in range(7):
                sh = 1 << k
                rolled = jnp.concatenate([a[M_ - sh:], a[:M_ - sh]], axis=0)
                a = jnp.where((il1 & sh) != 0, rolled, a)
            if c == 0:
                dsk_refs[c][b:b + 8 * M_:8, :] = a
            else:
                s = M_ - 128 * c
                dsk_refs[c][8 * 128 * c + b:8 * 128 * c + b + 8 * s:8, :] = \
                    a[:s]
                dsk_refs[c][b:b + 8 * (128 * c):8, :] = a[s:]

    idx = jax.lax.broadcasted_iota(jnp.int32, (B_BLK, N_), 1)

    def substep(p, dg, prev2, prev):
        c0 = jnp.where(p == 0, 0.0, LARGE_)
        r_nw = jnp.concatenate(
            [jnp.full((B_BLK, 1), c0, jnp.float32), prev2[:, :-1]], axis=1)
        r_n = jnp.concatenate(
            [jnp.full((B_BLK, 1), LARGE_, jnp.float32), prev[:, :-1]], axis=1)
        r_w = prev
        m = jnp.minimum(jnp.minimum(r_nw, r_n), r_w)
        rsum = (jnp.exp(m - r_nw) + jnp.exp(m - r_n) + jnp.exp(m - r_w))
        softmin = m - GAMMA_ * jnp.log(rsum)
        valid = (p - idx).astype(jnp.uint32) < jnp.uint32(M_)
        cur = jnp.where(valid, dg + softmin, LARGE_)
        return prev, cur

    def body(t, carry):
        prev2, prev = carry
        base = jnp.bitwise_and(t, 63) * 64
        slabs = [r[pl.ds(base, 64), :] for r in dsk_refs]
        p0 = t * 8
        for u in range(8):
            dg = jnp.concatenate(
                [s[8 * u:8 * (u + 1), :] for s in slabs], axis=1)
            prev2, prev = substep(p0 + u, dg, prev2, prev)
        return prev2, prev

    init = (jnp.full((B_BLK, N_), LARGE_, jnp.float32),
            jnp.full((B_BLK, N_), LARGE_, jnp.float32))
    last, _ = jax.lax.fori_loop(0, 128, body, init)
    out_ref[...] = last[None]


@jax.jit
def kernel(x, y):
    B = x.shape[0]
    nb = B // B_BLK
    out = pl.pallas_call(
        _sdtw_kernel,
        out_shape=jax.ShapeDtypeStruct((nb, B_BLK, N_), jnp.float32),
        grid=(nb,),
        in_specs=[
            pl.BlockSpec((B_BLK, N_, D_), lambda g: (g, 0, 0)),
            pl.BlockSpec((B_BLK, M_, D_), lambda g: (g, 0, 0)),
        ],
        out_specs=pl.BlockSpec((1, B_BLK, N_), lambda g: (g, 0, 0)),
        scratch_shapes=[
            pltpu.VMEM((8 * M_, 128), jnp.float32),
            pltpu.VMEM((8 * M_, 128), jnp.float32),
            pltpu.VMEM((8 * M_, 128), jnp.float32),
            pltpu.VMEM((8 * M_, 128), jnp.float32),
            pltpu.VMEM((M_, N_), jnp.float32),
        ],
        compiler_params=pltpu.CompilerParams(
            dimension_semantics=("parallel",),
            vmem_limit_bytes=56 * 1024 * 1024,
        ),
        name="soft_dtw_fused",
    )(x, y)
    return out[:, :, N_ - 1].reshape(B)


# speedup vs baseline: 19.4033x; 3.1889x over previous
import jax
import jax.numpy as jnp
from jax.experimental import pallas as pl
from jax.experimental.pallas import tpu as pltpu

GAMMA_ = 1.0
LARGE_ = 1e30
B_BLK = 32
NG = B_BLK // 8
N_ = 512
M_ = 512
D_ = 64


def _sdtw_kernel(x_ref, y_ref, out_ref, *refs):
    dsk_refs = refs[:4 * NG]
    dt_ref = refs[4 * NG]
    ones_row = jnp.ones((1, N_), jnp.float32)
    il1 = jax.lax.broadcasted_iota(jnp.int32, (1, 128), 1)

    for b in range(B_BLK):
        g, bb = b // 8, b % 8
        xbt = x_ref[b]
        ybt = y_ref[b]
        x2 = jnp.sum(xbt * xbt, axis=0, keepdims=True)
        y2 = jnp.sum(ybt * ybt, axis=0, keepdims=True)
        lhs = jnp.concatenate([-2.0 * ybt, y2, ones_row], axis=0)
        rhs = jnp.concatenate([xbt, ones_row, x2], axis=0)
        dt_ref[...] = jax.lax.dot_general(
            lhs, rhs, (((0,), (0,)), ((), ())),
            preferred_element_type=jnp.float32)
        for c in range(4):
            a = dt_ref[:, 128 * c:128 * (c + 1)]
            for k in range(7):
                sh = 1 << k
                rolled = jnp.concatenate([a[M_ - sh:], a[:M_ - sh]], axis=0)
                a = jnp.where((il1 & sh) != 0, rolled, a)
            dref = dsk_refs[4 * g + c]
            if c == 0:
                dref[bb:bb + 8 * M_:8, :] = a
            else:
                s = M_ - 128 * c
                dref[8 * 128 * c + bb:8 * 128 * c + bb + 8 * s:8, :] = a[:s]
                dref[bb:bb + 8 * (128 * c):8, :] = a[s:]

    idx = jax.lax.broadcasted_iota(jnp.int32, (B_BLK, N_), 1)
    inf_col = jnp.full((B_BLK, 1), LARGE_, jnp.float32)

    def substep(p, dg, carry):
        prev, psh, p2sh = carry
        m = jnp.minimum(jnp.minimum(p2sh, psh), prev)
        rsum = jnp.exp(m - p2sh) + jnp.exp(m - psh) + jnp.exp(m - prev)
        softmin = m - GAMMA_ * jnp.log(rsum)
        valid = (p - idx).astype(jnp.uint32) < jnp.uint32(M_)
        cur = jnp.where(valid, dg + softmin, LARGE_)
        cur_sh = jnp.concatenate([inf_col, cur[:, :-1]], axis=1)
        return cur, cur_sh, psh

    def diag_tile(slabs, u):
        return jnp.concatenate(
            [jnp.concatenate(
                [slabs[4 * g + c][8 * u:8 * (u + 1), :] for c in range(4)],
                axis=1) for g in range(NG)], axis=0)

    def read_slabs(base):
        return [r[pl.ds(base, 64), :] for r in dsk_refs]

    def body(t, carry):
        base = jnp.bitwise_and(t, 63) * 64
        slabs = read_slabs(base)
        p0 = t * 8
        for u in range(8):
            carry = substep(p0 + u, diag_tile(slabs, u), carry)
        return carry

    init = (jnp.full((B_BLK, N_), LARGE_, jnp.float32),
            jnp.full((B_BLK, N_), LARGE_, jnp.float32),
            jnp.where(idx == 0, 0.0, LARGE_))
    carry = jax.lax.fori_loop(0, 127, body, init)
    slabs = read_slabs(63 * 64)
    for u in range(0, 7):
        carry = substep(1016 + u, diag_tile(slabs, u), carry)
    out_ref[...] = carry[0][None]


@jax.jit
def kernel(x, y):
    B = x.shape[0]
    nb = B // B_BLK
    xt = jnp.swapaxes(x, 1, 2)
    yt = jnp.swapaxes(y, 1, 2)
    out = pl.pallas_call(
        _sdtw_kernel,
        out_shape=jax.ShapeDtypeStruct((nb, B_BLK, N_), jnp.float32),
        grid=(nb,),
        in_specs=[
            pl.BlockSpec((B_BLK, D_, N_), lambda g: (g, 0, 0)),
            pl.BlockSpec((B_BLK, D_, M_), lambda g: (g, 0, 0)),
        ],
        out_specs=pl.BlockSpec((1, B_BLK, N_), lambda g: (g, 0, 0)),
        scratch_shapes=(
            [pltpu.VMEM((8 * M_, 128), jnp.float32)] * (4 * NG)
            + [pltpu.VMEM((M_, N_), jnp.float32)]
        ),
        compiler_params=pltpu.CompilerParams(
            dimension_semantics=("parallel",),
            vmem_limit_bytes=56 * 1024 * 1024,
        ),
        name="soft_dtw_fused",
    )(xt, yt)
    return out[:, :, N_ - 1].reshape(B)


# speedup vs baseline: 28.0807x; 1.4472x over previous
import jax
import jax.numpy as jnp
from jax.experimental import pallas as pl
from jax.experimental.pallas import tpu as pltpu

GAMMA_ = 1.0
LARGE_ = 1e30
B_BLK = 32
NG = B_BLK // 8
N_ = 512
M_ = 512
D_ = 64


def _sdtw_kernel(x_ref, y_ref, out_ref, *refs):
    dsk_refs = refs[:4 * NG]
    dt_ref = refs[4 * NG]
    ones_row = jnp.ones((1, N_), jnp.float32)
    il1 = jax.lax.broadcasted_iota(jnp.int32, (1, 128), 1)

    for b in range(B_BLK):
        g, bb = b // 8, b % 8
        xbt = x_ref[b]
        ybt = y_ref[b]
        x2 = jnp.sum(xbt * xbt, axis=0, keepdims=True)
        y2 = jnp.sum(ybt * ybt, axis=0, keepdims=True)
        lhs = jnp.concatenate([-2.0 * ybt, y2, ones_row], axis=0)
        rhs = jnp.concatenate([xbt, ones_row, x2], axis=0)
        dt_ref[...] = jax.lax.dot_general(
            lhs, rhs, (((0,), (0,)), ((), ())),
            preferred_element_type=jnp.float32)
        for r in range(4):
            a = dt_ref[:, 128 * r:128 * (r + 1)]
            for k in range(7):
                sh = 4 << k
                rolled = jnp.concatenate([a[M_ - sh:], a[:M_ - sh]], axis=0)
                a = jnp.where((il1 & (1 << k)) != 0, rolled, a)
            dref = dsk_refs[4 * g + r]
            if r == 0:
                dref[bb:bb + 8 * M_:8, :] = a
            else:
                s = M_ - r
                dref[8 * r + bb:8 * r + bb + 8 * s:8, :] = a[:s]
                dref[bb:bb + 8 * r:8, :] = a[s:]

    idx4 = jax.lax.broadcasted_iota(jnp.int32, (B_BLK, 128), 1) * 4
    inf_col = jnp.full((B_BLK, 1), LARGE_, jnp.float32)
    one_col = jnp.ones((B_BLK, 1), jnp.float32)

    def shift_group(x3, fill_col):
        return jnp.concatenate([fill_col, x3[:, :-1]], axis=1)

    def substep(p, dg, carry):
        a, ash, a2sh, r, rsh, r2sh = carry
        an, rn = [], []
        for q in range(4):
            s = jnp.minimum(jnp.minimum(a2sh[q], ash[q]), a[q])
            rr = (r2sh[q] * jnp.exp(s - a2sh[q]) + rsh[q] * jnp.exp(s - ash[q])
                  + r[q] * jnp.exp(s - a[q]))
            valid = (p - (idx4 + q)).astype(jnp.uint32) < jnp.uint32(M_)
            an.append(jnp.where(valid, dg[q] + s, LARGE_))
            rn.append(jnp.where(valid, rr, 1.0))
        an_sh = [shift_group(an[3], inf_col), an[0], an[1], an[2]]
        rn_sh = [shift_group(rn[3], one_col), rn[0], rn[1], rn[2]]
        return an, an_sh, ash, rn, rn_sh, rsh

    def diag_tile(base, u):
        return [jnp.concatenate(
            [dsk_refs[4 * g + q][pl.ds(base + 8 * u, 8), :]
             for g in range(NG)], axis=0) for q in range(4)]

    def body(t, carry):
        base = jnp.bitwise_and(t, 63) * 64
        p0 = t * 8
        for u in range(8):
            carry = substep(p0 + u, diag_tile(base, u), carry)
        return carry

    large = jnp.full((B_BLK, 128), LARGE_, jnp.float32)
    ones = jnp.ones((B_BLK, 128), jnp.float32)
    lane0 = jax.lax.broadcasted_iota(jnp.int32, (B_BLK, 128), 1) == 0
    init = ([large] * 4, [large] * 4,
            [jnp.where(lane0, 0.0, LARGE_), large, large, large],
            [ones] * 4, [ones] * 4, [ones] * 4)
    carry = jax.lax.fori_loop(0, 127, body, init)
    for u in range(0, 7):
        carry = substep(1016 + u, diag_tile(63 * 64, u), carry)
    res3 = carry[0][3] - GAMMA_ * jnp.log(carry[3][3])
    out_ref[...] = jnp.concatenate(
        [carry[0][0], carry[0][1], carry[0][2], res3], axis=1)[None]


@jax.jit
def kernel(x, y):
    B = x.shape[0]
    nb = B // B_BLK
    xt = jnp.swapaxes(x, 1, 2).reshape(B, D_, 128, 4)
    xt = jnp.swapaxes(xt, 2, 3).reshape(B, D_, N_)
    yt = jnp.swapaxes(y, 1, 2)
    out = pl.pallas_call(
        _sdtw_kernel,
        out_shape=jax.ShapeDtypeStruct((nb, B_BLK, N_), jnp.float32),
        grid=(nb,),
        in_specs=[
            pl.BlockSpec((B_BLK, D_, N_), lambda g: (g, 0, 0)),
            pl.BlockSpec((B_BLK, D_, M_), lambda g: (g, 0, 0)),
        ],
        out_specs=pl.BlockSpec((1, B_BLK, N_), lambda g: (g, 0, 0)),
        scratch_shapes=(
            [pltpu.VMEM((8 * M_, 128), jnp.float32)] * (4 * NG)
            + [pltpu.VMEM((M_, N_), jnp.float32)]
        ),
        compiler_params=pltpu.CompilerParams(
            dimension_semantics=("parallel",),
            vmem_limit_bytes=56 * 1024 * 1024,
        ),
        name="soft_dtw_fused",
    )(xt, yt)
    return out[:, :, N_ - 1].reshape(B)
